# baseline (speedup 1.0000x reference)
import jax
import jax.numpy as jnp
from jax.experimental import pallas as pl
from jax.experimental.pallas import tpu as pltpu

_BP = 2184


def _make_block_fn(num_p):
    def _csd_block(conf_ref, cflip_ref, loc_ref, lflip_ref, out_ref):
        j = pl.program_id(1)
        c = conf_ref[0]
        cf = cflip_ref[0]
        l = loc_ref[0]
        lf = lflip_ref[0]

        bp = c.shape[0]
        row = j * bp + jax.lax.broadcasted_iota(jnp.int32, (bp, 1), 0)
        valid = row < num_p

        fg = jnp.max(c[:, 1:], axis=1, keepdims=True)
        mask = (fg > c[:, :1]) & valid

        p = c + 1e-7
        q = cf + 1e-7
        t = (p - q) * jnp.log(p / q)
        row_kl = jnp.sum(t, axis=1, keepdims=True)
        conf_p = jnp.sum(jnp.where(mask, row_kl, 0.0))

        d = l - lf
        row_loc = (jnp.sum(d * d, axis=1, keepdims=True)
                   + 4.0 * l[:, :1] * lf[:, :1])
        loc_p = jnp.sum(jnp.where(mask, row_loc, 0.0))

        cnt_p = jnp.sum(jnp.where(mask, 1.0, 0.0))

        partial = jnp.stack([cnt_p, conf_p, loc_p]).reshape(1, 3)

        first = (pl.program_id(0) == 0) & (j == 0)

        @pl.when(first)
        def _():
            out_ref[...] = partial

        @pl.when(jnp.logical_not(first))
        def _():
            out_ref[...] = out_ref[...] + partial

    return _csd_block


def kernel(conf, conf_flip, loc, loc_flip):
    b, num_p, c = conf.shape
    np_blocks = -(-num_p // _BP)

    grid = (b, np_blocks)
    out = pl.pallas_call(
        _make_block_fn(num_p),
        grid=grid,
        in_specs=[
            pl.BlockSpec((1, _BP, c), lambda i, j: (i, j, 0)),
            pl.BlockSpec((1, _BP, c), lambda i, j: (i, j, 0)),
            pl.BlockSpec((1, _BP, 4), lambda i, j: (i, j, 0)),
            pl.BlockSpec((1, _BP, 4), lambda i, j: (i, j, 0)),
        ],
        out_specs=pl.BlockSpec((1, 3), lambda i, j: (0, 0)),
        out_shape=jax.ShapeDtypeStruct((1, 3), jnp.float32),
        compiler_params=pltpu.CompilerParams(
            dimension_semantics=("arbitrary", "arbitrary"),
        ),
    )(conf, conf_flip, loc, loc_flip)

    cnt = jnp.maximum(out[0, 0], 1.0)
    return out[0, 1] / (2.0 * cnt) + out[0, 2] / (4.0 * cnt)
